# baseline (speedup 1.0000x reference)
import functools

import jax
import jax.numpy as jnp
from jax import lax
from jax.experimental import pallas as pl
from jax.experimental.pallas import tpu as pltpu
from jax.experimental.pallas import tpu_sc as plsc

NC = 2
NS = 16
NW = NC * NS

CHUNK = 800
NBUF = 2


@functools.partial(jax.jit, static_argnums=(2, 3))
def _sc_gather(table, idx, n, d):
    n_per_w = n // NW
    n_chunks = n_per_w // CHUNK
    assert n_chunks % NBUF == 0

    mesh = plsc.VectorSubcoreMesh(core_axis_name="c", subcore_axis_name="s")

    out_sds = jax.ShapeDtypeStruct((n, d), jnp.float32)

    @functools.partial(
        pl.kernel,
        mesh=mesh,
        out_type=(out_sds, out_sds),
        scratch_types=[
            pltpu.VMEM((NBUF, CHUNK), jnp.int32),
            pltpu.VMEM((NBUF, CHUNK, d), jnp.float32),
            [pltpu.SemaphoreType.DMA] * NBUF,
            [pltpu.SemaphoreType.DMA] * NBUF,
            [pltpu.SemaphoreType.DMA] * NBUF,
            [pltpu.SemaphoreType.DMA] * NBUF,
        ],
        compiler_params=pltpu.CompilerParams(use_tc_tiling_on_sc=False),
    )
    def k(table_hbm, idx_hbm, out0_hbm, out1_hbm, idx_v, rows_v,
          i_sems, g_sems, s0_sems, s1_sems):
        wid = lax.axis_index("s") * NC + lax.axis_index("c")
        base = wid * n_per_w

        def idx_copy(g, b):
            return pltpu.make_async_copy(
                idx_hbm.at[pl.ds(base + g * CHUNK, CHUNK)], idx_v.at[b],
                i_sems[b])

        def store_copy(g, b, out_hbm, sems):
            return pltpu.make_async_copy(
                rows_v.at[b], out_hbm.at[pl.ds(base + g * CHUNK, CHUNK)],
                sems[b])

        def chunk_step(g, b):
            @pl.when(g >= NBUF)
            def _():
                store_copy(g - NBUF, b, out0_hbm, s0_sems).wait()
                store_copy(g - NBUF, b, out1_hbm, s1_sems).wait()

            idx_copy(g, b).wait()
            gather = pltpu.make_async_copy(
                table_hbm.at[idx_v.at[b]], rows_v.at[b], g_sems[b])
            gather.start()

            @pl.when(g + 1 < n_chunks)
            def _():
                idx_copy(g + 1, 1 - b).start()

            gather.wait()
            store_copy(g, b, out0_hbm, s0_sems).start()
            store_copy(g, b, out1_hbm, s1_sems).start()

        idx_copy(0, 0).start()

        def body(p, carry):
            chunk_step(p * NBUF, 0)
            chunk_step(p * NBUF + 1, 1)
            return carry

        lax.fori_loop(0, n_chunks // NBUF, body, 0)
        for g, b in ((n_chunks - 2, 0), (n_chunks - 1, 1)):
            store_copy(g, b, out0_hbm, s0_sems).wait()
            store_copy(g, b, out1_hbm, s1_sems).wait()

    return k(table, idx)


def kernel(table, tensor, input_mask):
    v, d = table.shape
    b, l = tensor.shape
    n = b * l
    idx = tensor.reshape(n).astype(jnp.int32)
    emb0, emb1 = _sc_gather(table, idx, n, d)
    mod_mask = jnp.zeros((b, l), dtype=jnp.int32)
    return (emb0.reshape(b, l, d), emb1.reshape(b, l, d), input_mask, mod_mask)


# speedup vs baseline: 1.5048x; 1.5048x over previous
import functools

import jax
import jax.numpy as jnp
from jax import lax
from jax.experimental import pallas as pl
from jax.experimental.pallas import tpu as pltpu
from jax.experimental.pallas import tpu_sc as plsc

NC = 2
NS = 16
NW = NC * NS

CHUNK = 400
NBUF = 2


@functools.partial(jax.jit, static_argnums=(2,))
def _sc_gather(table128, idx, n):
    n_per_w = n // NW
    n_chunks = n_per_w // CHUNK

    mesh = plsc.VectorSubcoreMesh(core_axis_name="c", subcore_axis_name="s")

    @functools.partial(
        pl.kernel,
        mesh=mesh,
        out_type=jax.ShapeDtypeStruct((n, 128), jnp.float32),
        scratch_types=[
            [pltpu.VMEM((CHUNK,), jnp.int32)] * NBUF,
            [pltpu.VMEM((CHUNK, 128), jnp.float32)] * NBUF,
            [pltpu.SemaphoreType.DMA] * NBUF,
            [pltpu.SemaphoreType.DMA] * NBUF,
            [pltpu.SemaphoreType.DMA] * NBUF,
        ],
        compiler_params=pltpu.CompilerParams(use_tc_tiling_on_sc=True),
    )
    def k(table_hbm, idx_hbm, out_hbm, idx_v, rows_v, i_sems, g_sems, s_sems):
        wid = lax.axis_index("s") * NC + lax.axis_index("c")
        base = wid * n_per_w

        def idx_copy(g, b):
            return pltpu.make_async_copy(
                idx_hbm.at[pl.ds(base + g * CHUNK, CHUNK)], idx_v[b],
                i_sems[b])

        def store_copy(g, b):
            return pltpu.make_async_copy(
                rows_v[b], out_hbm.at[pl.ds(base + g * CHUNK, CHUNK)],
                s_sems[b])

        def chunk_step(g, b):
            @pl.when(g >= NBUF)
            def _():
                store_copy(g - NBUF, b).wait()

            idx_copy(g, b).wait()
            gather = pltpu.make_async_copy(
                table_hbm.at[idx_v[b]], rows_v[b], g_sems[b])
            gather.start()

            @pl.when(g + 1 < n_chunks)
            def _():
                idx_copy(g + 1, 1 - b).start()

            gather.wait()
            store_copy(g, b).start()

        idx_copy(0, 0).start()

        def body(p, carry):
            chunk_step(p * NBUF, 0)
            chunk_step(p * NBUF + 1, 1)
            return carry

        lax.fori_loop(0, n_chunks // NBUF, body, 0)
        store_copy(n_chunks - 2, 0).wait()
        store_copy(n_chunks - 1, 1).wait()

    return k(table128, idx)


def kernel(table, tensor, input_mask):
    v, d = table.shape
    b, l = tensor.shape
    n = b * l
    idx = tensor.reshape(n).astype(jnp.int32)
    table128 = jnp.pad(table, ((0, 0), (0, 128 - d)))
    emb = _sc_gather(table128, idx, n)[:, :d].reshape(b, l, d)
    mod_mask = jnp.zeros((b, l), dtype=jnp.int32)
    return (emb, emb, input_mask, mod_mask)
